# baseline (speedup 1.0000x reference)
import functools

import jax
import jax.numpy as jnp
from jax import lax
from jax.experimental import pallas as pl
from jax.experimental.pallas import tpu as pltpu
from jax.experimental.pallas import tpu_sc as plsc

N_E = 8192
E_DIM = 256
BETA = 0.25
B_TOK = 8192
T_BLK = 1024
T_STEPS = B_TOK // T_BLK
_SUB_SLICES = [
    (0, 1024, False), (1024, 1024, False), (2048, 688, True),
    (2736, 336, False), (3072, 1024, False), (4096, 1024, False),
    (5120, 352, True), (5472, 672, False), (6144, 1024, False),
    (7168, 1024, False),
]


def _lexmin(va, ia, vb, ib):
    take = (va < vb) | ((va == vb) & (ia < ib))
    return jnp.where(take, va, vb), jnp.where(take, ia, ib)


def _collapse(v, i):
    v, i = _lexmin(v[0:4], i[0:4], v[4:8], i[4:8])
    v, i = _lexmin(v[0:2], i[0:2], v[2:4], i[2:4])
    v, i = _lexmin(v[0:1], i[0:1], v[1:2], i[1:2])
    return v, i


def _dist_argmin_kernel(z_ref, en_ref, e2_ref, idx_ref, dsum_ref,
                        en2_scratch):
    t = pl.program_id(0)

    @pl.when(t == 0)
    def _():
        en2_scratch[...] = (en_ref[...] * 2.0).astype(jnp.bfloat16)

    zb = jnp.transpose(z_ref[0], (1, 0))
    zn_norm = jnp.sqrt(jnp.sum(zb * zb, axis=1, keepdims=True))
    zn = zb / jnp.maximum(zn_norm, 1e-12)
    z2 = jnp.sum(zn * zn, axis=1, keepdims=True)
    z2t = jnp.transpose(z2, (1, 0))
    zn16 = zn.astype(jnp.bfloat16)

    rowiota = lax.broadcasted_iota(jnp.int32, (8, T_BLK), 0)
    acc_v = jnp.full((8, T_BLK), jnp.inf, dtype=jnp.float32)
    acc_i = jnp.zeros((8, T_BLK), dtype=jnp.int32)
    for start, size, round_after in _SUB_SLICES:
        en2 = en2_scratch[pl.ds(start, size), :]
        e2s = e2_ref[pl.ds(start, size), :]
        s2 = lax.dot_general(en2, zn16, (((1,), (1,)), ((), ())),
                             preferred_element_type=jnp.float32)
        d = (z2t + e2s) - s2
        for r in range(size // 8):
            dv = d[r * 8:r * 8 + 8, :]
            iv = rowiota + (start + r * 8)
            take = dv < acc_v
            acc_v = jnp.where(take, dv, acc_v)
            acc_i = jnp.where(take, iv, acc_i)
        if round_after:
            v1, i1 = _collapse(acc_v, acc_i)
            v1 = v1.astype(jnp.bfloat16).astype(jnp.float32)
            acc_v = jnp.broadcast_to(v1, (8, T_BLK))
            acc_i = jnp.broadcast_to(i1, (8, T_BLK))

    v1, i1 = _collapse(acc_v, acc_i)
    idx_ref[...] = i1.reshape(1, 1, T_BLK)
    dsum_ref[...] = jnp.sum(v1).reshape(1, 1, 1)


def _dist_argmin(z_flat, en, e2):
    return pl.pallas_call(
        _dist_argmin_kernel,
        grid=(T_STEPS,),
        in_specs=[
            pl.BlockSpec((1, E_DIM, T_BLK), lambda t: (t, 0, 0)),
            pl.BlockSpec((N_E, E_DIM), lambda t: (0, 0)),
            pl.BlockSpec((N_E, 1), lambda t: (0, 0)),
        ],
        out_specs=[
            pl.BlockSpec((1, 1, T_BLK), lambda t: (t, 0, 0)),
            pl.BlockSpec((1, 1, 1), lambda t: (t, 0, 0)),
        ],
        out_shape=[
            jax.ShapeDtypeStruct((T_STEPS, 1, T_BLK), jnp.int32),
            jax.ShapeDtypeStruct((T_STEPS, 1, 1), jnp.float32),
        ],
        scratch_shapes=[pltpu.VMEM((N_E, E_DIM), jnp.bfloat16)],
    )(z_flat, en, e2)


@functools.cache
def _make_sc_gather():
    info = plsc.get_sparse_core_info()
    nw = info.num_cores * info.num_subcores
    b_per_w = B_TOK // nw

    @functools.partial(
        pl.kernel,
        out_type=jax.ShapeDtypeStruct((B_TOK, E_DIM), jnp.float32),
        mesh=plsc.VectorSubcoreMesh(core_axis_name="c", subcore_axis_name="s"),
        scratch_types=[
            pltpu.VMEM((b_per_w,), jnp.int32),
            pltpu.VMEM((b_per_w, E_DIM), jnp.float32),
            pltpu.SemaphoreType.DMA,
        ],
    )
    def _sc_gather(table_hbm, idx_hbm, out_hbm, idx_v, rows_v, sem):
        wid = lax.axis_index("s") * info.num_cores + lax.axis_index("c")
        base = wid * b_per_w
        pltpu.sync_copy(idx_hbm.at[pl.ds(base, b_per_w)], idx_v)
        pltpu.async_copy(table_hbm.at[idx_v], rows_v, sem).wait()
        pltpu.sync_copy(rows_v, out_hbm.at[pl.ds(base, b_per_w)])

    return _sc_gather


def kernel(z, embedding_weight):
    b, c, h, w = z.shape
    z_flat = z.reshape(b, E_DIM, h * w)
    e_norm = jnp.sqrt(jnp.sum(embedding_weight * embedding_weight,
                              axis=-1, keepdims=True))
    en = embedding_weight / jnp.maximum(e_norm, 1e-12)
    e2 = jnp.sum(en ** 2, axis=1)
    idx3, dsum = _dist_argmin(z_flat, en, e2.reshape(N_E, 1))
    indices = idx3.reshape(-1)
    zq_flat = _make_sc_gather()(en, indices)
    z_q = zq_flat.reshape(b, h, w, c).transpose(0, 3, 1, 2)
    m = jnp.sum(dsum) / jnp.float32(z.size)
    commit_loss = BETA * m + m
    return (z_q, commit_loss, indices)


# speedup vs baseline: 1.1250x; 1.1250x over previous
import functools

import jax
import jax.numpy as jnp
from jax import lax
from jax.experimental import pallas as pl
from jax.experimental.pallas import tpu as pltpu
from jax.experimental.pallas import tpu_sc as plsc

N_E = 8192
E_DIM = 256
BETA = 0.25
B_TOK = 8192
T_BLK = 2048
T_STEPS = B_TOK // T_BLK
_SUB_SLICES = [
    (0, 1024, False), (1024, 1024, False), (2048, 688, True),
    (2736, 336, False), (3072, 1024, False), (4096, 1024, False),
    (5120, 352, True), (5472, 672, False), (6144, 1024, False),
    (7168, 1024, False),
]


def _lexmin(va, ia, vb, ib):
    take = (va < vb) | ((va == vb) & (ia < ib))
    return jnp.where(take, va, vb), jnp.where(take, ia, ib)


def _collapse(v, i):
    v, i = _lexmin(v[0:4], i[0:4], v[4:8], i[4:8])
    v, i = _lexmin(v[0:2], i[0:2], v[2:4], i[2:4])
    v, i = _lexmin(v[0:1], i[0:1], v[1:2], i[1:2])
    return v, i


def _dist_argmin_kernel(z_ref, en_ref, e2_ref, idx_ref, dsum_ref,
                        en2_scratch):
    t = pl.program_id(0)

    @pl.when(t == 0)
    def _():
        en2_scratch[...] = (en_ref[...] * 2.0).astype(jnp.bfloat16)

    zb = z_ref[...]
    zn_norm = jnp.sqrt(jnp.sum(zb * zb, axis=1, keepdims=True))
    zn = zb / jnp.maximum(zn_norm, 1e-12)
    z2 = jnp.sum(zn * zn, axis=1, keepdims=True)
    z2t = jnp.transpose(z2, (1, 0))
    zn16 = zn.astype(jnp.bfloat16)

    rowiota = lax.broadcasted_iota(jnp.int32, (8, T_BLK), 0)
    acc_v = jnp.full((8, T_BLK), jnp.inf, dtype=jnp.float32)
    acc_i = jnp.zeros((8, T_BLK), dtype=jnp.int32)
    for start, size, round_after in _SUB_SLICES:
        en2 = en2_scratch[pl.ds(start, size), :]
        e2s = e2_ref[pl.ds(start, size), :]
        s2 = lax.dot_general(en2, zn16, (((1,), (1,)), ((), ())),
                             preferred_element_type=jnp.float32)
        for r in range(size // 8):
            dv = (z2t + e2s[r * 8:r * 8 + 8, :]) - s2[r * 8:r * 8 + 8, :]
            iv = rowiota + (start + r * 8)
            take = dv < acc_v
            acc_v = jnp.where(take, dv, acc_v)
            acc_i = jnp.where(take, iv, acc_i)
        if round_after:
            v1, i1 = _collapse(acc_v, acc_i)
            v1 = v1.astype(jnp.bfloat16).astype(jnp.float32)
            acc_v = jnp.broadcast_to(v1, (8, T_BLK))
            acc_i = jnp.broadcast_to(i1, (8, T_BLK))

    v1, i1 = _collapse(acc_v, acc_i)
    idx_ref[...] = i1.reshape(1, 1, T_BLK)
    dsum_ref[...] = jnp.sum(v1).reshape(1, 1, 1)


def _dist_argmin(z_flat, en, e2):
    return pl.pallas_call(
        _dist_argmin_kernel,
        grid=(T_STEPS,),
        in_specs=[
            pl.BlockSpec((T_BLK, E_DIM), lambda t: (t, 0)),
            pl.BlockSpec((N_E, E_DIM), lambda t: (0, 0)),
            pl.BlockSpec((N_E, 1), lambda t: (0, 0)),
        ],
        out_specs=[
            pl.BlockSpec((1, 1, T_BLK), lambda t: (t, 0, 0)),
            pl.BlockSpec((1, 1, 1), lambda t: (t, 0, 0)),
        ],
        out_shape=[
            jax.ShapeDtypeStruct((T_STEPS, 1, T_BLK), jnp.int32),
            jax.ShapeDtypeStruct((T_STEPS, 1, 1), jnp.float32),
        ],
        scratch_shapes=[pltpu.VMEM((N_E, E_DIM), jnp.bfloat16)],
    )(z_flat, en, e2)


@functools.cache
def _make_sc_gather():
    info = plsc.get_sparse_core_info()
    nw = info.num_cores * info.num_subcores
    b_per_w = B_TOK // nw

    @functools.partial(
        pl.kernel,
        out_type=jax.ShapeDtypeStruct((B_TOK, E_DIM), jnp.float32),
        mesh=plsc.VectorSubcoreMesh(core_axis_name="c", subcore_axis_name="s"),
        scratch_types=[
            pltpu.VMEM((b_per_w,), jnp.int32),
            pltpu.VMEM((b_per_w, E_DIM), jnp.float32),
            pltpu.SemaphoreType.DMA,
        ],
    )
    def _sc_gather(table_hbm, idx_hbm, out_hbm, idx_v, rows_v, sem):
        wid = lax.axis_index("s") * info.num_cores + lax.axis_index("c")
        base = wid * b_per_w
        pltpu.sync_copy(idx_hbm.at[pl.ds(base, b_per_w)], idx_v)
        pltpu.async_copy(table_hbm.at[idx_v], rows_v, sem).wait()
        pltpu.sync_copy(rows_v, out_hbm.at[pl.ds(base, b_per_w)])

    return _sc_gather


def kernel(z, embedding_weight):
    b, c, h, w = z.shape
    z_flat = jnp.transpose(z, (0, 2, 3, 1)).reshape(-1, E_DIM)
    e_norm = jnp.sqrt(jnp.sum(embedding_weight * embedding_weight,
                              axis=-1, keepdims=True))
    en = embedding_weight / jnp.maximum(e_norm, 1e-12)
    e2 = jnp.sum(en ** 2, axis=1)
    idx3, dsum = _dist_argmin(z_flat, en, e2.reshape(N_E, 1))
    indices = idx3.reshape(-1)
    zq_flat = _make_sc_gather()(en, indices)
    z_q = zq_flat.reshape(b, h, w, c).transpose(0, 3, 1, 2)
    m = jnp.sum(dsum) / jnp.float32(z.size)
    commit_loss = BETA * m + m
    return (z_q, commit_loss, indices)
